# baseline (speedup 1.0000x reference)
import functools

import jax
import jax.numpy as jnp
from jax import lax
from jax.experimental import pallas as pl
from jax.experimental.pallas import tpu as pltpu
from jax.experimental.pallas import tpu_sc as plsc

_DIM = 16
_BATCH = 16384
_N_SCORE = 32
_NUM_ITEMS = 100000

_NC, _NS = 2, 16
_NW = _NC * _NS
_B_W = _BATCH // _NW
_CHUNK = 128
_N_CHUNK = _B_W // _CHUNK

_BI = 12800


def _sc_distances(user_emb, item_emb, user_ids, pos_ids, neg_ids):
    mesh = plsc.VectorSubcoreMesh(core_axis_name="c", subcore_axis_name="s")

    @functools.partial(
        pl.kernel,
        mesh=mesh,
        compiler_params=pltpu.CompilerParams(
            use_tc_tiling_on_sc=False, needs_layout_passes=False),
        out_type=[
            jax.ShapeDtypeStruct((_BATCH,), jnp.float32),
            jax.ShapeDtypeStruct((_BATCH,), jnp.float32),
        ],
        scratch_types=[
            pltpu.VMEM((_N_CHUNK, _CHUNK), jnp.int32),
            pltpu.VMEM((_N_CHUNK, _CHUNK), jnp.int32),
            pltpu.VMEM((_N_CHUNK, _CHUNK), jnp.int32),
            pltpu.VMEM((_B_W, _DIM), jnp.float32),
            pltpu.VMEM((_B_W, _DIM), jnp.float32),
            pltpu.VMEM((_B_W, _DIM), jnp.float32),
            pltpu.VMEM((_B_W,), jnp.float32),
            pltpu.VMEM((_B_W,), jnp.float32),
            pltpu.SemaphoreType.DMA,
        ],
    )
    def k(user_hbm, item_hbm, uid_hbm, pid_hbm, nid_hbm,
          pos_hbm, neg_hbm,
          uid_v, pid_v, nid_v, u_v, p_v, n_v, pos_v, neg_v, sem):
        wid = lax.axis_index("s") * _NC + lax.axis_index("c")
        base = wid * _B_W

        for c in range(_N_CHUNK):
            off = base + c * _CHUNK
            pltpu.sync_copy(uid_hbm.at[pl.ds(off, _CHUNK)], uid_v.at[c])
            pltpu.sync_copy(pid_hbm.at[pl.ds(off, _CHUNK)], pid_v.at[c])
            pltpu.sync_copy(nid_hbm.at[pl.ds(off, _CHUNK)], nid_v.at[c])

        copies = []
        for c in range(_N_CHUNK):
            dst = pl.ds(c * _CHUNK, _CHUNK)
            copies.append(pltpu.async_copy(user_hbm.at[uid_v.at[c]], u_v.at[dst], sem))
            copies.append(pltpu.async_copy(item_hbm.at[pid_v.at[c]], p_v.at[dst], sem))
            copies.append(pltpu.async_copy(item_hbm.at[nid_v.at[c]], n_v.at[dst], sem))
        for cp in copies:
            cp.wait()

        lane = lax.iota(jnp.int32, 16)

        def body(g, carry):
            accp = jnp.zeros((16,), jnp.float32)
            accn = jnp.zeros((16,), jnp.float32)
            for rr in range(16):
                r = g * 16 + rr
                u = u_v[r, :]
                dp = u - p_v[r, :]
                dn = u - n_v[r, :]
                ps = jnp.sum(dp * dp)
                ns = jnp.sum(dn * dn)
                m = lane == rr
                accp = jnp.where(m, ps, accp)
                accn = jnp.where(m, ns, accn)
            pos_v[pl.ds(g * 16, 16)] = accp
            neg_v[pl.ds(g * 16, 16)] = accn
            return carry

        lax.fori_loop(0, _B_W // 16, body, 0, unroll=1)

        pltpu.sync_copy(pos_v, pos_hbm.at[pl.ds(base, _B_W)])
        pltpu.sync_copy(neg_v, neg_hbm.at[pl.ds(base, _B_W)])

    return k(user_emb, item_emb, user_ids, pos_ids, neg_ids)


def _tc_scores(user_emb, item_emb, score_ids):
    def body(sid_ref, user_ref, it_ref, out_ref, su_ref, sem):
        i = pl.program_id(0)

        @pl.when(i == 0)
        def _():
            copies = []
            for j in range(_N_SCORE):
                copies.append(pltpu.make_async_copy(
                    user_ref.at[pl.ds(sid_ref[j], 1)],
                    su_ref.at[pl.ds(j, 1)], sem))
            for cp in copies:
                cp.start()
            for cp in copies:
                cp.wait()

        su = su_ref[...]
        it = it_ref[...]
        dots = lax.dot_general(su, it, (((1,), (1,)), ((), ())),
                               preferred_element_type=jnp.float32)
        su2 = jnp.sum(su * su, axis=1)
        it2 = jnp.sum(it * it, axis=1)
        out_ref[...] = 2.0 * dots - su2[:, None] - it2[None, :]

    return pl.pallas_call(
        body,
        grid=(pl.cdiv(_NUM_ITEMS, _BI),),
        in_specs=[
            pl.BlockSpec(memory_space=pltpu.SMEM),
            pl.BlockSpec(memory_space=pl.ANY),
            pl.BlockSpec((_BI, _DIM), lambda i: (i, 0)),
        ],
        out_specs=pl.BlockSpec((_N_SCORE, _BI), lambda i: (0, i)),
        out_shape=jax.ShapeDtypeStruct((_N_SCORE, _NUM_ITEMS), jnp.float32),
        scratch_shapes=[
            pltpu.VMEM((_N_SCORE, _DIM), jnp.float32),
            pltpu.SemaphoreType.DMA,
        ],
    )(score_ids, user_emb, item_emb)


def kernel(user_embeddings, item_embeddings, user_ids, pos_item_ids,
           neg_item_ids, score_user_ids):
    pos_d, neg_d = _sc_distances(
        user_embeddings, item_embeddings, user_ids, pos_item_ids, neg_item_ids)
    scores = _tc_scores(user_embeddings, item_embeddings, score_user_ids)
    return (pos_d, neg_d, scores)
